# baseline (speedup 1.0000x reference)
import functools

import jax
import jax.numpy as jnp
from jax import lax
from jax.experimental import pallas as pl
from jax.experimental.pallas import tpu as pltpu
from jax.experimental.pallas import tpu_sc as plsc

ALPHA = 0.7
TAU_SENT = 1.0

_NC = 2
_NS = 16
_L = 16


def _sc_gather(x1_hbm, t_hbm, out_hbm, t_v, flat_v, val_v, sem):
    bpw = t_v.shape[0]
    wid = lax.axis_index("s") * _NC + lax.axis_index("c")
    base = wid * bpw
    V = x1_hbm.shape[0] // t_hbm.shape[0]

    pltpu.sync_copy(t_hbm.at[pl.ds(base, bpw)], t_v)
    for j in range(bpw // _L):
        r16 = lax.iota(jnp.int32, _L) + (base + j * _L)
        flat_v[pl.ds(j * _L, _L)] = r16 * V + t_v[pl.ds(j * _L, _L)]
    pltpu.async_copy(x1_hbm.at[flat_v], val_v, sem).wait()
    pltpu.sync_copy(val_v, out_hbm.at[pl.ds(base, bpw)])


def _loss_body(x_ref, t_ref, m_ref, tv_ref, ml_ref, tot_ref, acc_ref):
    i = pl.program_id(0)
    nb = pl.num_programs(0)
    x = x_ref[0]
    t = t_ref[0]
    m = m_ref[0]
    tv = tv_ref[0]
    S, V = x.shape

    iota = lax.broadcasted_iota(jnp.int32, (S, V), 1)
    maxv = jnp.max(x, axis=1, keepdims=True)
    idx = jnp.min(jnp.where(x == maxv, iota, V), axis=1,
                  keepdims=True)
    match = (idx == t).astype(jnp.float32)

    sent = jnp.exp(jnp.sum(match, axis=0, keepdims=True)
                   * (1.0 / (S * TAU_SENT)))
    mlp = jnp.sum(tv * m, axis=0, keepdims=True)
    msp = jnp.sum(m, axis=0, keepdims=True)
    outp = sent * jnp.sum(maxv * m, axis=0, keepdims=True)
    denp = sent * msp

    @pl.when(i == 0)
    def _init():
        acc_ref[...] = jnp.zeros_like(acc_ref)

    acc_ref[0:1, 0:1] += mlp
    acc_ref[1:2, 0:1] += msp
    acc_ref[2:3, 0:1] += outp
    acc_ref[3:4, 0:1] += denp

    @pl.when(i == nb - 1)
    def _finish():
        ml = -acc_ref[0:1, 0:1] / acc_ref[1:2, 0:1]
        out = -acc_ref[2:3, 0:1] / acc_ref[3:4, 0:1]
        ml_ref[...] = ml
        tot_ref[...] = ALPHA * out + (1.0 - ALPHA) * ml


def kernel(input, target, mask):
    B, S, V = input.shape
    R = B * S
    t_flat = target.astype(jnp.int32).reshape(R)
    bpw = R // (_NC * _NS)

    sc = pl.kernel(
        functools.partial(_sc_gather),
        out_type=jax.ShapeDtypeStruct((R,), jnp.float32),
        mesh=plsc.VectorSubcoreMesh(core_axis_name="c", subcore_axis_name="s"),
        scratch_types=[
            pltpu.VMEM((bpw,), jnp.int32),
            pltpu.VMEM((bpw,), jnp.int32),
            pltpu.VMEM((bpw,), jnp.float32),
            pltpu.SemaphoreType.DMA,
        ],
    )
    tvals = sc(input.reshape(R * V), t_flat)

    t3 = t_flat.reshape(B, S, 1)
    m3 = mask.astype(jnp.float32).reshape(B, S, 1)
    tv3 = tvals.reshape(B, S, 1)

    ml, tot = pl.pallas_call(
        _loss_body,
        grid=(B,),
        in_specs=[
            pl.BlockSpec((1, S, V), lambda i: (i, 0, 0)),
            pl.BlockSpec((1, S, 1), lambda i: (i, 0, 0)),
            pl.BlockSpec((1, S, 1), lambda i: (i, 0, 0)),
            pl.BlockSpec((1, S, 1), lambda i: (i, 0, 0)),
        ],
        out_specs=[
            pl.BlockSpec((1, 1), lambda i: (0, 0)),
            pl.BlockSpec((1, 1), lambda i: (0, 0)),
        ],
        out_shape=[
            jax.ShapeDtypeStruct((1, 1), jnp.float32),
            jax.ShapeDtypeStruct((1, 1), jnp.float32),
        ],
        scratch_shapes=[pltpu.VMEM((8, 128), jnp.float32)],
    )(input, t3, m3, tv3)
    return (ml.reshape(()), tot.reshape(()))


# speedup vs baseline: 1.8995x; 1.8995x over previous
import jax
import jax.numpy as jnp
from jax import lax
from jax.experimental import pallas as pl
from jax.experimental.pallas import tpu as pltpu

ALPHA = 0.7
TAU_SENT = 1.0
_W = 128


def _loss_body(ts_ref, x_ref, t_ref, m_ref, xany_ref, ml_ref, tot_ref,
               grow_ref, acc_ref, sem):
    i = pl.program_id(0)
    nb = pl.num_programs(0)
    t = t_ref[0]
    m = m_ref[0]
    _, S, V = x_ref.shape

    copies = []
    for s in range(S):
        start = pl.multiple_of(ts_ref[i, s], _W)
        cp = pltpu.make_async_copy(
            xany_ref.at[i, pl.ds((s // 8) * 8, 8), pl.ds(start, _W)],
            grow_ref.at[s],
            sem,
        )
        cp.start()
        copies.append(cp)

    x = x_ref[0]
    iota = lax.broadcasted_iota(jnp.int32, (S, V), 1).astype(jnp.float32)
    tf = t.astype(jnp.float32)
    maxv = jnp.max(x, axis=1, keepdims=True)
    idxf = jnp.min(jnp.where(x == maxv, iota, 3.4e38), axis=1,
                   keepdims=True)
    match = (idxf == tf).astype(jnp.float32)

    for cp in copies:
        cp.wait()
    sub = lax.broadcasted_iota(jnp.int32, (S, 8, _W), 1)
    row = lax.broadcasted_iota(jnp.int32, (S, 8, _W), 0)
    lane = lax.broadcasted_iota(jnp.int32, (S, 8, _W), 2)
    pick = (sub == (row & 7)) & (lane == (t.reshape(S, 1, 1) & (_W - 1)))
    z = jnp.sum(jnp.where(pick, grow_ref[...], 0.0), axis=2)
    tval = jnp.sum(z, axis=1, keepdims=True)

    sent = jnp.exp(jnp.sum(match, axis=0, keepdims=True)
                   * (1.0 / (S * TAU_SENT)))
    mlp = jnp.sum(tval * m, axis=0, keepdims=True)
    msp = jnp.sum(m, axis=0, keepdims=True)
    outp = sent * jnp.sum(maxv * m, axis=0, keepdims=True)
    denp = sent * msp

    @pl.when(i == 0)
    def _init():
        acc_ref[...] = jnp.zeros_like(acc_ref)

    acc_ref[0:1, 0:1] += mlp
    acc_ref[1:2, 0:1] += msp
    acc_ref[2:3, 0:1] += outp
    acc_ref[3:4, 0:1] += denp

    @pl.when(i == nb - 1)
    def _finish():
        ml = -acc_ref[0:1, 0:1] / acc_ref[1:2, 0:1]
        out = -acc_ref[2:3, 0:1] / acc_ref[3:4, 0:1]
        ml_ref[...] = ml
        tot_ref[...] = ALPHA * out + (1.0 - ALPHA) * ml


def kernel(input, target, mask):
    B, S, V = input.shape
    t32 = target.astype(jnp.int32)
    t3 = t32.reshape(B, S, 1)
    m3 = mask.astype(jnp.float32).reshape(B, S, 1)
    tstart = (t32 // _W) * _W

    ml, tot = pl.pallas_call(
        _loss_body,
        grid=(B,),
        in_specs=[
            pl.BlockSpec(memory_space=pltpu.SMEM),
            pl.BlockSpec((1, S, V), lambda i: (i, 0, 0)),
            pl.BlockSpec((1, S, 1), lambda i: (i, 0, 0)),
            pl.BlockSpec((1, S, 1), lambda i: (i, 0, 0)),
            pl.BlockSpec(memory_space=pl.ANY),
        ],
        out_specs=[
            pl.BlockSpec((1, 1), lambda i: (0, 0)),
            pl.BlockSpec((1, 1), lambda i: (0, 0)),
        ],
        out_shape=[
            jax.ShapeDtypeStruct((1, 1), jnp.float32),
            jax.ShapeDtypeStruct((1, 1), jnp.float32),
        ],
        scratch_shapes=[
            pltpu.VMEM((S, 8, _W), jnp.float32),
            pltpu.VMEM((8, 128), jnp.float32),
            pltpu.SemaphoreType.DMA,
        ],
    )(tstart, input, t3, m3, input)
    return (ml.reshape(()), tot.reshape(()))


# speedup vs baseline: 2.3923x; 1.2594x over previous
import jax
import jax.numpy as jnp
from jax import lax
from jax.experimental import pallas as pl
from jax.experimental.pallas import tpu as pltpu

ALPHA = 0.7
TAU_SENT = 1.0
_W = 128


def _loss_body(ts_ref, x_ref, t_ref, m_ref, *rest):
    grefs = rest[:-3]
    ml_ref, tot_ref, acc_ref = rest[-3:]
    i = pl.program_id(0)
    nb = pl.num_programs(0)
    t = t_ref[0]
    m = m_ref[0]
    _, S, V = x_ref.shape

    x = x_ref[0]
    iota = lax.broadcasted_iota(jnp.int32, (S, V), 1).astype(jnp.float32)
    tf = t.astype(jnp.float32)
    maxv = jnp.max(x, axis=1, keepdims=True)
    idxf = jnp.min(jnp.where(x == maxv, iota, 3.4e38), axis=1,
                   keepdims=True)
    match = (idxf == tf).astype(jnp.float32)

    lane2 = lax.broadcasted_iota(jnp.int32, (1, _W), 1)
    tvs = []
    for s in range(S):
        vs = grefs[s][0, pl.ds(s % 8, 1), :]
        tmod = ts_ref[1, i, s]
        tvs.append(jnp.sum(jnp.where(lane2 == tmod, vs, 0.0), axis=1,
                           keepdims=True))
    tval = jnp.concatenate(tvs, axis=0)

    sent = jnp.exp(jnp.sum(match, axis=0, keepdims=True)
                   * (1.0 / (S * TAU_SENT)))
    mlp = jnp.sum(tval * m, axis=0, keepdims=True)
    msp = jnp.sum(m, axis=0, keepdims=True)
    outp = sent * jnp.sum(maxv * m, axis=0, keepdims=True)
    denp = sent * msp

    @pl.when(i == 0)
    def _init():
        acc_ref[...] = jnp.zeros_like(acc_ref)

    acc_ref[0:1, 0:1] += mlp
    acc_ref[1:2, 0:1] += msp
    acc_ref[2:3, 0:1] += outp
    acc_ref[3:4, 0:1] += denp

    @pl.when(i == nb - 1)
    def _finish():
        ml = -acc_ref[0:1, 0:1] / acc_ref[1:2, 0:1]
        out = -acc_ref[2:3, 0:1] / acc_ref[3:4, 0:1]
        ml_ref[...] = ml
        tot_ref[...] = ALPHA * out + (1.0 - ALPHA) * ml


def kernel(input, target, mask):
    B, S, V = input.shape
    t32 = target.astype(jnp.int32)
    t3 = t32.reshape(B, S, 1)
    m3 = mask.astype(jnp.float32).reshape(B, S, 1)
    tpre = jnp.stack([t32 // _W, t32 % _W])

    def gmap(s):
        return lambda i, ts: (i, s // 8, ts[0, i, s])

    gspecs = [pl.BlockSpec((1, 8, _W), gmap(s)) for s in range(S)]

    grid_spec = pltpu.PrefetchScalarGridSpec(
        num_scalar_prefetch=1,
        grid=(B,),
        in_specs=[
            pl.BlockSpec((1, S, V), lambda i, ts: (i, 0, 0)),
            pl.BlockSpec((1, S, 1), lambda i, ts: (i, 0, 0)),
            pl.BlockSpec((1, S, 1), lambda i, ts: (i, 0, 0)),
        ] + gspecs,
        out_specs=[
            pl.BlockSpec((1, 1), lambda i, ts: (0, 0)),
            pl.BlockSpec((1, 1), lambda i, ts: (0, 0)),
        ],
        scratch_shapes=[pltpu.VMEM((8, 128), jnp.float32)],
    )

    ml, tot = pl.pallas_call(
        _loss_body,
        grid_spec=grid_spec,
        out_shape=[
            jax.ShapeDtypeStruct((1, 1), jnp.float32),
            jax.ShapeDtypeStruct((1, 1), jnp.float32),
        ],
    )(tpre, input, t3, m3, *([input] * S))
    return (ml.reshape(()), tot.reshape(()))


# speedup vs baseline: 2.7313x; 1.1417x over previous
import jax
import jax.numpy as jnp
from jax import lax
from jax.experimental import pallas as pl
from jax.experimental.pallas import tpu as pltpu

ALPHA = 0.7
TAU_SENT = 1.0
_W = 128


def _loss_body(ts_ref, x_ref, t_ref, m_ref, ml_ref, tot_ref, acc_ref):
    i = pl.program_id(0)
    nb = pl.num_programs(0)
    t = t_ref[0]
    m = m_ref[0]
    _, S, V = x_ref.shape

    x = x_ref[0]
    iota = lax.broadcasted_iota(jnp.int32, (S, V), 1).astype(jnp.float32)
    tf = t.astype(jnp.float32)
    maxv = jnp.max(x, axis=1, keepdims=True)
    idxf = jnp.min(jnp.where(x == maxv, iota, 3.4e38), axis=1,
                   keepdims=True)
    match = (idxf == tf).astype(jnp.float32)

    lane2 = lax.broadcasted_iota(jnp.int32, (1, _W), 1)
    tvs = []
    for s in range(S):
        start = pl.multiple_of(ts_ref[i, s], _W)
        vs = x_ref[0, pl.ds(s, 1), pl.ds(start, _W)]
        tmod = ts_ref[i + nb, s]
        tvs.append(jnp.sum(jnp.where(lane2 == tmod, vs, 0.0), axis=1,
                           keepdims=True))
    tval = jnp.concatenate(tvs, axis=0)

    sent = jnp.exp(jnp.sum(match, axis=0, keepdims=True)
                   * (1.0 / (S * TAU_SENT)))
    mlp = jnp.sum(tval * m, axis=0, keepdims=True)
    msp = jnp.sum(m, axis=0, keepdims=True)
    outp = sent * jnp.sum(maxv * m, axis=0, keepdims=True)
    denp = sent * msp

    @pl.when(i == 0)
    def _init():
        acc_ref[...] = jnp.zeros_like(acc_ref)

    acc_ref[0:1, 0:1] += mlp
    acc_ref[1:2, 0:1] += msp
    acc_ref[2:3, 0:1] += outp
    acc_ref[3:4, 0:1] += denp

    @pl.when(i == nb - 1)
    def _finish():
        ml = -acc_ref[0:1, 0:1] / acc_ref[1:2, 0:1]
        out = -acc_ref[2:3, 0:1] / acc_ref[3:4, 0:1]
        ml_ref[...] = ml
        tot_ref[...] = ALPHA * out + (1.0 - ALPHA) * ml


def kernel(input, target, mask):
    B, S, V = input.shape
    t32 = target.astype(jnp.int32)
    t3 = t32.reshape(B, S, 1)
    m3 = mask.astype(jnp.float32).reshape(B, S, 1)
    tpre = jnp.concatenate([(t32 // _W) * _W, t32 % _W], axis=0)

    ml, tot = pl.pallas_call(
        _loss_body,
        grid=(B,),
        in_specs=[
            pl.BlockSpec(memory_space=pltpu.SMEM),
            pl.BlockSpec((1, S, V), lambda i: (i, 0, 0)),
            pl.BlockSpec((1, S, 1), lambda i: (i, 0, 0)),
            pl.BlockSpec((1, S, 1), lambda i: (i, 0, 0)),
        ],
        out_specs=[
            pl.BlockSpec((1, 1), lambda i: (0, 0)),
            pl.BlockSpec((1, 1), lambda i: (0, 0)),
        ],
        out_shape=[
            jax.ShapeDtypeStruct((1, 1), jnp.float32),
            jax.ShapeDtypeStruct((1, 1), jnp.float32),
        ],
        scratch_shapes=[pltpu.VMEM((8, 128), jnp.float32)],
    )(tpre, input, t3, m3)
    return (ml.reshape(()), tot.reshape(()))
